# baseline (speedup 1.0000x reference)
import functools

import jax
import jax.numpy as jnp
import numpy as np
from jax.experimental import pallas as pl
from jax.experimental.pallas import tpu as pltpu

H = 768
F = 3072
E = 8
TOPK = 2
R = 16
LORA_SCALE = 2.0

TB = 512
NCH = 4
FC = F // NCH

_NOISE_CACHE = {}


def _fixed_noise(shape):
    if shape not in _NOISE_CACHE:
        try:
            with jax.ensure_compile_time_eval():
                val = jax.random.normal(jax.random.key(42), shape,
                                        dtype=jnp.float32)
                _NOISE_CACHE[shape] = np.asarray(val)
        except Exception:
            return None
    return _NOISE_CACHE[shape]


def _fused_kernel(x_ref, wr_ref, wn_ref, nz_ref, a3_ref, b3_ref,
                  wg_hbm, wu_hbm, wd_hbm, out_ref, rl_ref,
                  m_s, b_s, wg_f, wu_f, wd_f, wg_b, wu_b, wd_b,
                  sem_g, sem_u, sem_d):
    i = pl.program_id(0)
    c = pl.program_id(1)

    def g_copy(cc):
        sl = pl.ds(cc * FC, FC)
        return pltpu.make_async_copy(wg_hbm.at[:, sl], wg_f.at[:, sl],
                                     sem_g.at[cc])

    def u_copy(cc):
        sl = pl.ds(cc * FC, FC)
        return pltpu.make_async_copy(wu_hbm.at[:, sl], wu_f.at[:, sl],
                                     sem_u.at[cc])

    def d_copy(cc):
        sl = pl.ds(cc * FC, FC)
        return pltpu.make_async_copy(wd_hbm.at[sl, :], wd_f.at[sl, :],
                                     sem_d.at[cc])

    @pl.when((i == 0) & (c == 0))
    def _start():
        m_s[:, 0:E] = wr_ref[...].astype(jnp.bfloat16)
        m_s[:, E:2 * E] = wn_ref[...].astype(jnp.bfloat16)
        for e in range(E):
            m_s[:, 2 * E + e * R:2 * E + (e + 1) * R] = (
                a3_ref[e].astype(jnp.bfloat16))
            b_s[e * R:(e + 1) * R, :] = b3_ref[e].astype(jnp.bfloat16)
        for cc in range(NCH):
            g_copy(cc).start()
            u_copy(cc).start()
            d_copy(cc).start()

    csl = pl.ds(c * FC, FC)

    @pl.when(i == 0)
    def _wait_and_pack_chunk():
        g_copy(c).wait()
        u_copy(c).wait()
        d_copy(c).wait()
        wg_b[:, csl] = wg_f[:, csl].astype(jnp.bfloat16)
        wu_b[:, csl] = wu_f[:, csl].astype(jnp.bfloat16)
        wd_b[csl, :] = wd_f[csl, :].astype(jnp.bfloat16)

    x = x_ref[0].astype(jnp.bfloat16)

    gate = jnp.dot(x, wg_b[:, csl], preferred_element_type=jnp.float32)
    up = jnp.dot(x, wu_b[:, csl], preferred_element_type=jnp.float32)
    h = (jax.nn.silu(gate) * up).astype(jnp.bfloat16)
    partial = jnp.dot(h, wd_b[csl, :], preferred_element_type=jnp.float32)

    @pl.when(c == 0)
    def _router_and_lora():
        r = jnp.dot(x, m_s[...], preferred_element_type=jnp.float32)
        logits = r[:, 0:E]
        nlog = r[:, E:2 * E]
        t = r[:, 2 * E:]
        rl = logits + nz_ref[...] * jax.nn.softplus(nlog)
        rl_ref[...] = rl

        p = jax.nn.softmax(rl, axis=-1)
        iota = jax.lax.broadcasted_iota(jnp.int32, p.shape, 1)
        m1 = jnp.max(p, axis=-1, keepdims=True)
        a1 = jnp.min(jnp.where(p == m1, iota, E), axis=-1, keepdims=True)
        mask1 = iota == a1
        p2 = jnp.where(mask1, -jnp.inf, p)
        m2 = jnp.max(p2, axis=-1, keepdims=True)
        a2 = jnp.min(jnp.where(p2 == m2, iota, E), axis=-1, keepdims=True)
        mask2 = iota == a2
        dw = (jnp.where(mask1, m1, 0.0) + jnp.where(mask2, m2, 0.0)) / (m1 + m2)

        erow = jax.lax.broadcasted_iota(jnp.int32, (E, E * R), 0)
        ecol = jax.lax.broadcasted_iota(jnp.int32, (E, E * R), 1) // R
        expand = (erow == ecol).astype(jnp.float32)
        w_rep = jnp.dot(dw, expand, preferred_element_type=jnp.float32)

        lora = jnp.dot((t * w_rep).astype(jnp.bfloat16), b_s[...],
                       preferred_element_type=jnp.float32) * LORA_SCALE
        out_ref[0] = lora + partial

    @pl.when(c != 0)
    def _accumulate():
        out_ref[0] = out_ref[0] + partial


@functools.partial(jax.jit, static_argnames=())
def _run(hs, w_route, w_noise, noise, lora_a, lora_b, w_gate, w_up, w_down):
    S = hs.shape[1]
    NT = S // TB
    grid = (NT, NCH)
    out, rl = pl.pallas_call(
        _fused_kernel,
        grid=grid,
        in_specs=[
            pl.BlockSpec((1, TB, H), lambda i, c: (0, i, 0)),
            pl.BlockSpec((H, E), lambda i, c: (0, 0)),
            pl.BlockSpec((H, E), lambda i, c: (0, 0)),
            pl.BlockSpec((TB, E), lambda i, c: (i, 0)),
            pl.BlockSpec((E, H, R), lambda i, c: (0, 0, 0)),
            pl.BlockSpec((E, R, H), lambda i, c: (0, 0, 0)),
            pl.BlockSpec(memory_space=pltpu.MemorySpace.HBM),
            pl.BlockSpec(memory_space=pltpu.MemorySpace.HBM),
            pl.BlockSpec(memory_space=pltpu.MemorySpace.HBM),
        ],
        out_specs=[
            pl.BlockSpec((1, TB, H), lambda i, c: (0, i, 0)),
            pl.BlockSpec((TB, E), lambda i, c: (i, 0)),
        ],
        out_shape=[
            jax.ShapeDtypeStruct((1, S, H), jnp.float32),
            jax.ShapeDtypeStruct((S, E), jnp.float32),
        ],
        scratch_shapes=[
            pltpu.VMEM((H, 2 * E + E * R), jnp.bfloat16),
            pltpu.VMEM((E * R, H), jnp.bfloat16),
            pltpu.VMEM((H, F), jnp.float32),
            pltpu.VMEM((H, F), jnp.float32),
            pltpu.VMEM((F, H), jnp.float32),
            pltpu.VMEM((H, F), jnp.bfloat16),
            pltpu.VMEM((H, F), jnp.bfloat16),
            pltpu.VMEM((F, H), jnp.bfloat16),
            pltpu.SemaphoreType.DMA((NCH,)),
            pltpu.SemaphoreType.DMA((NCH,)),
            pltpu.SemaphoreType.DMA((NCH,)),
        ],
        compiler_params=pltpu.CompilerParams(
            vmem_limit_bytes=100 * 1024 * 1024,
        ),
    )(hs, w_route, w_noise, noise, lora_a, lora_b, w_gate, w_up, w_down)
    return out, rl


def kernel(hidden_states, w_route, w_noise, lora_a, lora_b, w_gate, w_up, w_down):
    B, S, Hd = hidden_states.shape
    noise = _fixed_noise((B * S, E))
    if noise is None:
        noise = jax.random.normal(jax.random.key(42), (B * S, E),
                                  dtype=jnp.float32)
    out, rl = _run(hidden_states, w_route, w_noise, noise, lora_a, lora_b,
                   w_gate, w_up, w_down)
    return out, rl


# speedup vs baseline: 1.0694x; 1.0694x over previous
import functools

import jax
import jax.numpy as jnp
import numpy as np
from jax.experimental import pallas as pl
from jax.experimental.pallas import tpu as pltpu

H = 768
F = 3072
E = 8
TOPK = 2
R = 16
LORA_SCALE = 2.0

TB = 1024
NCH = 4
FC = F // NCH

_NOISE_CACHE = {}


def _fixed_noise(shape):
    if shape not in _NOISE_CACHE:
        try:
            with jax.ensure_compile_time_eval():
                val = jax.random.normal(jax.random.key(42), shape,
                                        dtype=jnp.float32)
                _NOISE_CACHE[shape] = np.asarray(val)
        except Exception:
            return None
    return _NOISE_CACHE[shape]


def _fused_kernel(x_ref, wr_ref, wn_ref, nz_ref, a3_ref, b3_ref,
                  wg_hbm, wu_hbm, wd_hbm, out_ref, rl_ref,
                  m_s, b_s, wg_v, wu_v, wd_v,
                  sem_g, sem_u, sem_d):
    i = pl.program_id(0)
    c = pl.program_id(1)

    def g_copy(cc):
        sl = pl.ds(cc * FC, FC)
        return pltpu.make_async_copy(wg_hbm.at[:, sl], wg_v.at[:, sl],
                                     sem_g.at[cc])

    def u_copy(cc):
        sl = pl.ds(cc * FC, FC)
        return pltpu.make_async_copy(wu_hbm.at[:, sl], wu_v.at[:, sl],
                                     sem_u.at[cc])

    def d_copy(cc):
        sl = pl.ds(cc * FC, FC)
        return pltpu.make_async_copy(wd_hbm.at[sl, :], wd_v.at[sl, :],
                                     sem_d.at[cc])

    @pl.when((i == 0) & (c == 0))
    def _start():
        m_s[:, 0:E] = wr_ref[...]
        m_s[:, E:2 * E] = wn_ref[...]
        for e in range(E):
            m_s[:, 2 * E + e * R:2 * E + (e + 1) * R] = a3_ref[e]
            b_s[e * R:(e + 1) * R, :] = b3_ref[e]
        for cc in range(NCH):
            g_copy(cc).start()
            u_copy(cc).start()
            d_copy(cc).start()

    @pl.when(i == 0)
    def _wait_chunk():
        g_copy(c).wait()
        u_copy(c).wait()
        d_copy(c).wait()

    x = x_ref[0]
    csl = pl.ds(c * FC, FC)

    gate = jnp.dot(x, wg_v[:, csl], preferred_element_type=jnp.float32)
    up = jnp.dot(x, wu_v[:, csl], preferred_element_type=jnp.float32)
    h = jax.nn.silu(gate) * up
    partial = jnp.dot(h, wd_v[csl, :], preferred_element_type=jnp.float32)

    @pl.when(c == 0)
    def _router_and_lora():
        r = jnp.dot(x, m_s[...], preferred_element_type=jnp.float32)
        logits = r[:, 0:E]
        nlog = r[:, E:2 * E]
        t = r[:, 2 * E:]
        rl = logits + nz_ref[...] * jax.nn.softplus(nlog)
        rl_ref[...] = rl

        p = jax.nn.softmax(rl, axis=-1)
        iota = jax.lax.broadcasted_iota(jnp.int32, p.shape, 1)
        m1 = jnp.max(p, axis=-1, keepdims=True)
        a1 = jnp.min(jnp.where(p == m1, iota, E), axis=-1, keepdims=True)
        mask1 = iota == a1
        p2 = jnp.where(mask1, -jnp.inf, p)
        m2 = jnp.max(p2, axis=-1, keepdims=True)
        a2 = jnp.min(jnp.where(p2 == m2, iota, E), axis=-1, keepdims=True)
        mask2 = iota == a2
        dw = (jnp.where(mask1, m1, 0.0) + jnp.where(mask2, m2, 0.0)) / (m1 + m2)

        erow = jax.lax.broadcasted_iota(jnp.int32, (E, E * R), 0)
        ecol = jax.lax.broadcasted_iota(jnp.int32, (E, E * R), 1) // R
        expand = (erow == ecol).astype(jnp.float32)
        w_rep = jnp.dot(dw, expand, preferred_element_type=jnp.float32)

        lora = jnp.dot(t * w_rep, b_s[...],
                       preferred_element_type=jnp.float32) * LORA_SCALE
        out_ref[0] = lora + partial

    @pl.when(c != 0)
    def _accumulate():
        out_ref[0] = out_ref[0] + partial


@functools.partial(jax.jit, static_argnames=())
def _run(hs, w_route, w_noise, noise, lora_a, lora_b, w_gate, w_up, w_down):
    S = hs.shape[1]
    NT = S // TB
    grid = (NT, NCH)
    out, rl = pl.pallas_call(
        _fused_kernel,
        grid=grid,
        in_specs=[
            pl.BlockSpec((1, TB, H), lambda i, c: (0, i, 0)),
            pl.BlockSpec((H, E), lambda i, c: (0, 0)),
            pl.BlockSpec((H, E), lambda i, c: (0, 0)),
            pl.BlockSpec((TB, E), lambda i, c: (i, 0)),
            pl.BlockSpec((E, H, R), lambda i, c: (0, 0, 0)),
            pl.BlockSpec((E, R, H), lambda i, c: (0, 0, 0)),
            pl.BlockSpec(memory_space=pltpu.MemorySpace.HBM),
            pl.BlockSpec(memory_space=pltpu.MemorySpace.HBM),
            pl.BlockSpec(memory_space=pltpu.MemorySpace.HBM),
        ],
        out_specs=[
            pl.BlockSpec((1, TB, H), lambda i, c: (0, i, 0)),
            pl.BlockSpec((TB, E), lambda i, c: (i, 0)),
        ],
        out_shape=[
            jax.ShapeDtypeStruct((1, S, H), jnp.float32),
            jax.ShapeDtypeStruct((S, E), jnp.float32),
        ],
        scratch_shapes=[
            pltpu.VMEM((H, 2 * E + E * R), jnp.float32),
            pltpu.VMEM((E * R, H), jnp.float32),
            pltpu.VMEM((H, F), jnp.float32),
            pltpu.VMEM((H, F), jnp.float32),
            pltpu.VMEM((F, H), jnp.float32),
            pltpu.SemaphoreType.DMA((NCH,)),
            pltpu.SemaphoreType.DMA((NCH,)),
            pltpu.SemaphoreType.DMA((NCH,)),
        ],
        compiler_params=pltpu.CompilerParams(
            vmem_limit_bytes=100 * 1024 * 1024,
        ),
    )(hs, w_route, w_noise, noise, lora_a, lora_b, w_gate, w_up, w_down)
    return out, rl


def kernel(hidden_states, w_route, w_noise, lora_a, lora_b, w_gate, w_up, w_down):
    B, S, Hd = hidden_states.shape
    noise = _fixed_noise((B * S, E))
    if noise is None:
        noise = jax.random.normal(jax.random.key(42), (B * S, E),
                                  dtype=jnp.float32)
    out, rl = _run(hidden_states, w_route, w_noise, noise, lora_a, lora_b,
                   w_gate, w_up, w_down)
    return out, rl
